# Initial kernel scaffold; baseline (speedup 1.0000x reference)
import jax
import jax.numpy as jnp
from jax.experimental import pallas as pl


def kernel(output, tokens, scores):
    raise NotImplementedError("write your pallas kernel here")


# speedup vs baseline: 34.7179x; 34.7179x over previous
import jax
import jax.numpy as jnp
from jax import lax
from jax.experimental import pallas as pl

BEAM = 8
VOCAB = 100000
END_ID = 2
NEG_INF = float("-inf")
IMAX = 2**31 - 1


def _body(out_ref, tok_ref, sc_ref, best_ref, ntok_ref, done_ref):
    lp = out_ref[:, :]
    s = lp + sc_ref[:, :]
    row = lax.broadcasted_iota(jnp.int32, (BEAM, VOCAB), 0)
    col = lax.broadcasted_iota(jnp.int32, (BEAM, VOCAB), 1)
    flat = row * VOCAB + col

    vals = []
    idxs = []
    for _ in range(BEAM):
        m = jnp.max(s)
        cand = jnp.where(s == m, flat, IMAX)
        idx = jnp.min(cand)
        vals.append(m)
        idxs.append(idx)
        s = jnp.where(flat == idx, NEG_INF, s)

    rows81 = lax.broadcasted_iota(jnp.int32, (BEAM, 1), 0)
    rows82 = lax.broadcasted_iota(jnp.int32, (BEAM, 2), 0)
    cols82 = lax.broadcasted_iota(jnp.int32, (BEAM, 2), 1)

    best = jnp.zeros((BEAM, 1), jnp.float32)
    ntok = jnp.zeros((BEAM, 2), jnp.int32)
    word0 = None
    for i in range(BEAM):
        beam_i = idxs[i] // VOCAB
        word_i = idxs[i] % VOCAB
        if i == 0:
            word0 = word_i
        gath_i = jnp.sum(jnp.where(rows81 == beam_i, tok_ref[:, :], 0))
        best = jnp.where(rows81 == i, vals[i], best)
        ntok = jnp.where(rows82 == i,
                         jnp.where(cols82 == 0, gath_i, word_i), ntok)

    best_ref[:, :] = best
    ntok_ref[:, :] = ntok
    done_ref[:, :] = jnp.full((1, 1), (word0 == END_ID).astype(jnp.int32))


def kernel(output, tokens, scores):
    lp = output[:, -1, :]
    best, ntok, done = pl.pallas_call(
        _body,
        grid=(),
        in_specs=[
            pl.BlockSpec((BEAM, VOCAB), lambda: (0, 0)),
            pl.BlockSpec((BEAM, 1), lambda: (0, 0)),
            pl.BlockSpec((BEAM, 1), lambda: (0, 0)),
        ],
        out_specs=[
            pl.BlockSpec((BEAM, 1), lambda: (0, 0)),
            pl.BlockSpec((BEAM, 2), lambda: (0, 0)),
            pl.BlockSpec((1, 1), lambda: (0, 0)),
        ],
        out_shape=[
            jax.ShapeDtypeStruct((BEAM, 1), jnp.float32),
            jax.ShapeDtypeStruct((BEAM, 2), jnp.int32),
            jax.ShapeDtypeStruct((1, 1), jnp.int32),
        ],
    )(lp, tokens, scores)
    return best, ntok, (done[0, 0] == 1)
